# baseline (speedup 1.0000x reference)
import jax
import jax.numpy as jnp
from jax import lax
from jax.experimental import pallas as pl
from jax.experimental.pallas import tpu as pltpu
from jax.experimental.pallas import tpu_sc as plsc

N_NODES = 10000
N_EDGES = 160000
D_FEAT = 256
HALF = 128

NC = 2
NS = 16
CHUNK = 40
EDGES_PER_TILE = N_EDGES // NS
N_CHUNKS = EDGES_PER_TILE // CHUNK
NRING = 3
FLUSH_TILES = 10
FLUSH_ROWS = N_NODES // FLUSH_TILES


def _pq_body(nodes_ref, w_ref, out_ref):
    out_ref[0, 0] = jnp.dot(nodes_ref[...], w_ref[0],
                            preferred_element_type=jnp.float32)


def _compute_pq(nodes, we_sd):
    blk = 2000
    grid = (N_NODES // blk, 2, 2)
    return pl.pallas_call(
        _pq_body,
        grid=grid,
        in_specs=[
            pl.BlockSpec((blk, D_FEAT), lambda i, w, c: (i, 0)),
            pl.BlockSpec((1, D_FEAT, HALF), lambda i, w, c: (w, 0, c)),
        ],
        out_specs=pl.BlockSpec((1, 1, blk, HALF), lambda i, w, c: (w, c, i, 0)),
        out_shape=jax.ShapeDtypeStruct((2, 2, N_NODES, HALF), jnp.float32),
    )(nodes, we_sd)


def _r_body(e_ref, w_ref, b_ref, out_ref):
    out_ref[0] = (jnp.dot(e_ref[...], w_ref[...],
                          preferred_element_type=jnp.float32)
                  + b_ref[...][None, :])


def _compute_r(edges, we_e, b_e):
    blk = 4000
    grid = (N_EDGES // blk, 2)
    return pl.pallas_call(
        _r_body,
        grid=grid,
        in_specs=[
            pl.BlockSpec((blk, 16), lambda j, c: (j, 0)),
            pl.BlockSpec((16, HALF), lambda j, c: (0, c)),
            pl.BlockSpec((HALF,), lambda j, c: (c,)),
        ],
        out_specs=pl.BlockSpec((1, blk, HALF), lambda j, c: (c, j, 0)),
        out_shape=jax.ShapeDtypeStruct((2, N_EDGES, HALF), jnp.float32),
    )(edges, we_e, b_e)


def _node_body(nodes_ref, agg_ref, wn1_ref, wn2_ref, b_ref, out_ref):
    acc = jnp.dot(nodes_ref[...], wn1_ref[...],
                  preferred_element_type=jnp.float32)
    acc += jnp.dot(agg_ref[0], wn2_ref[0], preferred_element_type=jnp.float32)
    acc += jnp.dot(agg_ref[1], wn2_ref[1], preferred_element_type=jnp.float32)
    out_ref[...] = jnp.maximum(acc + b_ref[...][None, :], 0.0)


def _node_update(nodes, agg, wn1, wn2r, b_n):
    blk = 2000
    grid = (N_NODES // blk, 2)
    return pl.pallas_call(
        _node_body,
        grid=grid,
        in_specs=[
            pl.BlockSpec((blk, D_FEAT), lambda i, h: (i, 0)),
            pl.BlockSpec((2, blk, HALF), lambda i, h: (0, i, 0)),
            pl.BlockSpec((D_FEAT, HALF), lambda i, h: (0, h)),
            pl.BlockSpec((2, HALF, HALF), lambda i, h: (0, 0, h)),
            pl.BlockSpec((HALF,), lambda i, h: (h,)),
        ],
        out_specs=pl.BlockSpec((blk, HALF), lambda i, h: (i, h)),
        out_shape=jax.ShapeDtypeStruct((N_NODES, D_FEAT), jnp.float32),
    )(nodes, agg, wn1, wn2r, b_n)


def _sc_body(src3_hbm, dst3_hbm, p0_hbm, p1_hbm, q0_hbm, q1_hbm,
             r0_hbm, r1_hbm, z_hbm,
             enew_hbm, agg_hbm,
             idxs, idxd,
             pbuf0, pbuf1, pbuf2,
             qbuf0, qbuf1, qbuf2,
             rbuf0, rbuf1, rbuf2,
             acc,
             sem_i0, sem_i1, sem_i2,
             sem_g0, sem_g1, sem_g2,
             sem_o0, sem_o1, sem_o2):
    c = lax.axis_index("c")
    s = lax.axis_index("s")
    pbufs = (pbuf0, pbuf1, pbuf2)
    qbufs = (qbuf0, qbuf1, qbuf2)
    rbufs = (rbuf0, rbuf1, rbuf2)
    sems_i = (sem_i0, sem_i1, sem_i2)
    sems_g = (sem_g0, sem_g1, sem_g2)
    sems_o = (sem_o0, sem_o1, sem_o2)

    @pl.when(s < FLUSH_TILES)
    def _zero():
        pltpu.sync_copy(z_hbm.at[pl.ds(s * FLUSH_ROWS, FLUSH_ROWS)],
                        acc.at[pl.ds(s * FLUSH_ROWS, FLUSH_ROWS)])
    plsc.subcore_barrier()

    def issue_idx(j, r):
        pltpu.async_copy(src3_hbm.at[s, j], idxs.at[r], sems_i[r])
        pltpu.async_copy(dst3_hbm.at[s, j], idxd.at[r], sems_i[r])

    def wait_idx(r):
        pltpu.make_async_copy(src3_hbm.at[s, 0], idxs.at[r],
                              sems_i[r]).wait()
        pltpu.make_async_copy(dst3_hbm.at[s, 0], idxd.at[r],
                              sems_i[r]).wait()

    def issue_gathers(j, r):
        base = s * EDGES_PER_TILE + j * CHUNK

        @pl.when(c == 0)
        def _c0():
            pltpu.async_copy(p0_hbm.at[idxs.at[r]], pbufs[r], sems_g[r])
            pltpu.async_copy(q0_hbm.at[idxd.at[r]], qbufs[r], sems_g[r])
            pltpu.async_copy(r0_hbm.at[pl.ds(base, CHUNK)], rbufs[r],
                             sems_g[r])

        @pl.when(c == 1)
        def _c1():
            pltpu.async_copy(p1_hbm.at[idxs.at[r]], pbufs[r], sems_g[r])
            pltpu.async_copy(q1_hbm.at[idxd.at[r]], qbufs[r], sems_g[r])
            pltpu.async_copy(r1_hbm.at[pl.ds(base, CHUNK)], rbufs[r],
                             sems_g[r])

    def wait_gathers(r):
        @pl.when(c == 0)
        def _c0():
            pltpu.make_async_copy(p0_hbm.at[idxs.at[r]], pbufs[r],
                                  sems_g[r]).wait()
            pltpu.make_async_copy(q0_hbm.at[idxd.at[r]], qbufs[r],
                                  sems_g[r]).wait()
            pltpu.make_async_copy(r0_hbm.at[pl.ds(0, CHUNK)], rbufs[r],
                                  sems_g[r]).wait()

        @pl.when(c == 1)
        def _c1():
            pltpu.make_async_copy(p1_hbm.at[idxs.at[r]], pbufs[r],
                                  sems_g[r]).wait()
            pltpu.make_async_copy(q1_hbm.at[idxd.at[r]], qbufs[r],
                                  sems_g[r]).wait()
            pltpu.make_async_copy(r1_hbm.at[pl.ds(0, CHUNK)], rbufs[r],
                                  sems_g[r]).wait()

    def process(j, r):
        pbuf, qbuf, rbuf = pbufs[r], qbufs[r], rbufs[r]
        wait_gathers(r)

        def row(rw, cr):
            for g in range(HALF // 16):
                sl = pl.ds(g * 16, 16)
                pbuf[rw, sl] = jnp.maximum(
                    pbuf[rw, sl] + qbuf[rw, sl] + rbuf[rw, sl], 0.0)
            return cr

        lax.fori_loop(0, CHUNK, row, 0, unroll=2)
        base = s * EDGES_PER_TILE + j * CHUNK
        pltpu.async_copy(
            pbuf, enew_hbm.at[pl.ds(base, CHUNK), pl.ds(c * HALF, HALF)],
            sems_o[r])
        pltpu.sync_copy(pbuf, acc.at[idxd.at[r]], add=True)

    def drain_out(r):
        pltpu.make_async_copy(
            pbufs[r], enew_hbm.at[pl.ds(0, CHUNK), pl.ds(c * HALF, HALF)],
            sems_o[r]).wait()

    def slot(t, rt):
        @pl.when(t + 1 < N_CHUNKS)
        def _gathers_next():
            wait_idx((rt + 1) % NRING)
            issue_gathers(t + 1, (rt + 1) % NRING)

        @pl.when(t + 2 < N_CHUNKS)
        def _idx_ahead():
            @pl.when(t >= 1)
            def _drain():
                drain_out((rt + 2) % NRING)
            issue_idx(t + 2, (rt + 2) % NRING)

        @pl.when(t < N_CHUNKS)
        def _proc():
            process(t, rt)

    issue_idx(0, 0)
    issue_idx(1, 1)
    wait_idx(0)
    issue_gathers(0, 0)

    def triple(m, carry):
        slot(3 * m, 0)
        slot(3 * m + 1, 1)
        slot(3 * m + 2, 2)
        return carry

    lax.fori_loop(0, (N_CHUNKS + 2) // 3, triple, 0)
    drain_out(0)
    drain_out(1)
    drain_out(2)
    plsc.subcore_barrier()

    @pl.when(s < FLUSH_TILES)
    def _flush():
        pltpu.sync_copy(acc.at[pl.ds(s * FLUSH_ROWS, FLUSH_ROWS)],
                        agg_hbm.at[c, pl.ds(s * FLUSH_ROWS, FLUSH_ROWS)])


def _sc_edge_pass(src3, dst3, p0, p1, q0, q1, r0, r1, zeros):
    mesh = plsc.VectorSubcoreMesh(core_axis_name="c", subcore_axis_name="s")
    f = pl.kernel(
        _sc_body,
        mesh=mesh,
        out_type=[
            jax.ShapeDtypeStruct((N_EDGES, D_FEAT), jnp.float32),
            jax.ShapeDtypeStruct((2, N_NODES, HALF), jnp.float32),
        ],
        scratch_types=(
            [pltpu.VMEM((NRING, CHUNK), jnp.int32)] * 2
            + [pltpu.VMEM((CHUNK, HALF), jnp.float32)] * 9
            + [pltpu.VMEM_SHARED((N_NODES, HALF), jnp.float32)]
            + [pltpu.SemaphoreType.DMA] * 9
        ),
    )
    return f(src3, dst3, p0, p1, q0, q1, r0, r1, zeros)


def kernel(nodes, edges, edge_index, W_e, b_e, W_n, b_n):
    src3 = edge_index[0].reshape(NS, N_CHUNKS, CHUNK)
    dst3 = edge_index[1].reshape(NS, N_CHUNKS, CHUNK)
    we_sd = jnp.stack([W_e[:D_FEAT], W_e[D_FEAT:2 * D_FEAT]])
    we_e = W_e[2 * D_FEAT:]
    wn1 = W_n[:D_FEAT]
    wn2r = W_n[D_FEAT:].reshape(2, HALF, D_FEAT)

    pq = _compute_pq(nodes, we_sd)
    r3 = _compute_r(edges, we_e, b_e)
    zeros = jnp.zeros((N_NODES, HALF), jnp.float32)

    e_new, agg = _sc_edge_pass(src3, dst3, pq[0, 0], pq[0, 1],
                               pq[1, 0], pq[1, 1], r3[0], r3[1], zeros)
    n_new = _node_update(nodes, agg, wn1, wn2r, b_n)
    return (n_new, e_new)


# speedup vs baseline: 1.0026x; 1.0026x over previous
import jax
import jax.numpy as jnp
from jax import lax
from jax.experimental import pallas as pl
from jax.experimental.pallas import tpu as pltpu
from jax.experimental.pallas import tpu_sc as plsc

N_NODES = 10000
N_EDGES = 160000
D_FEAT = 256
HALF = 128

NC = 2
NS = 16
CHUNK = 40
EDGES_PER_TILE = N_EDGES // NS
N_CHUNKS = EDGES_PER_TILE // CHUNK
NRING = 3
FLUSH_TILES = 10
FLUSH_ROWS = N_NODES // FLUSH_TILES


def _pq_body(nodes_ref, w_ref, out_ref):
    out_ref[0, 0] = jnp.dot(nodes_ref[...], w_ref[0],
                            preferred_element_type=jnp.float32)


def _compute_pq(nodes, we_sd):
    blk = 2000
    grid = (N_NODES // blk, 2, 2)
    return pl.pallas_call(
        _pq_body,
        grid=grid,
        in_specs=[
            pl.BlockSpec((blk, D_FEAT), lambda i, w, c: (i, 0)),
            pl.BlockSpec((1, D_FEAT, HALF), lambda i, w, c: (w, 0, c)),
        ],
        out_specs=pl.BlockSpec((1, 1, blk, HALF), lambda i, w, c: (w, c, i, 0)),
        out_shape=jax.ShapeDtypeStruct((2, 2, N_NODES, HALF), jnp.float32),
    )(nodes, we_sd)


def _r_body(e_ref, w_ref, b_ref, out_ref):
    out_ref[0] = (jnp.dot(e_ref[...], w_ref[...],
                          preferred_element_type=jnp.float32)
                  + b_ref[...][None, :])


def _compute_r(edges, we_e, b_e):
    blk = 4000
    grid = (N_EDGES // blk, 2)
    return pl.pallas_call(
        _r_body,
        grid=grid,
        in_specs=[
            pl.BlockSpec((blk, 16), lambda j, c: (j, 0)),
            pl.BlockSpec((16, HALF), lambda j, c: (0, c)),
            pl.BlockSpec((HALF,), lambda j, c: (c,)),
        ],
        out_specs=pl.BlockSpec((1, blk, HALF), lambda j, c: (c, j, 0)),
        out_shape=jax.ShapeDtypeStruct((2, N_EDGES, HALF), jnp.float32),
    )(edges, we_e, b_e)


def _node_body(nodes_ref, agg_ref, wn1_ref, wn2_ref, b_ref, out_ref):
    acc = jnp.dot(nodes_ref[...], wn1_ref[...],
                  preferred_element_type=jnp.float32)
    acc += jnp.dot(agg_ref[0], wn2_ref[0], preferred_element_type=jnp.float32)
    acc += jnp.dot(agg_ref[1], wn2_ref[1], preferred_element_type=jnp.float32)
    out_ref[...] = jnp.maximum(acc + b_ref[...][None, :], 0.0)


def _node_update(nodes, agg, wn1, wn2r, b_n):
    blk = 2000
    grid = (N_NODES // blk, 2)
    return pl.pallas_call(
        _node_body,
        grid=grid,
        in_specs=[
            pl.BlockSpec((blk, D_FEAT), lambda i, h: (i, 0)),
            pl.BlockSpec((2, blk, HALF), lambda i, h: (0, i, 0)),
            pl.BlockSpec((D_FEAT, HALF), lambda i, h: (0, h)),
            pl.BlockSpec((2, HALF, HALF), lambda i, h: (0, 0, h)),
            pl.BlockSpec((HALF,), lambda i, h: (h,)),
        ],
        out_specs=pl.BlockSpec((blk, HALF), lambda i, h: (i, h)),
        out_shape=jax.ShapeDtypeStruct((N_NODES, D_FEAT), jnp.float32),
    )(nodes, agg, wn1, wn2r, b_n)


def _sc_body(src3_hbm, dst3_hbm, p0_hbm, p1_hbm, q0_hbm, q1_hbm,
             r0_hbm, r1_hbm, z_hbm,
             enew_hbm, agg_hbm,
             idxs, idxd,
             pbuf0, pbuf1, pbuf2,
             qbuf0, qbuf1, qbuf2,
             rbuf0, rbuf1, rbuf2,
             acc,
             sem_i0, sem_i1, sem_i2,
             sem_g0, sem_g1, sem_g2,
             sem_o0, sem_o1, sem_o2,
             sem_a0, sem_a1, sem_a2):
    c = lax.axis_index("c")
    s = lax.axis_index("s")
    pbufs = (pbuf0, pbuf1, pbuf2)
    qbufs = (qbuf0, qbuf1, qbuf2)
    rbufs = (rbuf0, rbuf1, rbuf2)
    sems_i = (sem_i0, sem_i1, sem_i2)
    sems_g = (sem_g0, sem_g1, sem_g2)
    sems_o = (sem_o0, sem_o1, sem_o2)
    sems_a = (sem_a0, sem_a1, sem_a2)

    @pl.when(s < FLUSH_TILES)
    def _zero():
        pltpu.sync_copy(z_hbm.at[pl.ds(s * FLUSH_ROWS, FLUSH_ROWS)],
                        acc.at[pl.ds(s * FLUSH_ROWS, FLUSH_ROWS)])
    plsc.subcore_barrier()

    def issue_idx(j, r):
        pltpu.async_copy(src3_hbm.at[s, j], idxs.at[r], sems_i[r])
        pltpu.async_copy(dst3_hbm.at[s, j], idxd.at[r], sems_i[r])

    def wait_idx(r):
        pltpu.make_async_copy(src3_hbm.at[s, 0], idxs.at[r],
                              sems_i[r]).wait()
        pltpu.make_async_copy(dst3_hbm.at[s, 0], idxd.at[r],
                              sems_i[r]).wait()

    def issue_gathers(j, r):
        base = s * EDGES_PER_TILE + j * CHUNK

        @pl.when(c == 0)
        def _c0():
            pltpu.async_copy(p0_hbm.at[idxs.at[r]], pbufs[r], sems_g[r])
            pltpu.async_copy(q0_hbm.at[idxd.at[r]], qbufs[r], sems_g[r])
            pltpu.async_copy(r0_hbm.at[pl.ds(base, CHUNK)], rbufs[r],
                             sems_g[r])

        @pl.when(c == 1)
        def _c1():
            pltpu.async_copy(p1_hbm.at[idxs.at[r]], pbufs[r], sems_g[r])
            pltpu.async_copy(q1_hbm.at[idxd.at[r]], qbufs[r], sems_g[r])
            pltpu.async_copy(r1_hbm.at[pl.ds(base, CHUNK)], rbufs[r],
                             sems_g[r])

    def wait_gathers(r):
        @pl.when(c == 0)
        def _c0():
            pltpu.make_async_copy(p0_hbm.at[idxs.at[r]], pbufs[r],
                                  sems_g[r]).wait()
            pltpu.make_async_copy(q0_hbm.at[idxd.at[r]], qbufs[r],
                                  sems_g[r]).wait()
            pltpu.make_async_copy(r0_hbm.at[pl.ds(0, CHUNK)], rbufs[r],
                                  sems_g[r]).wait()

        @pl.when(c == 1)
        def _c1():
            pltpu.make_async_copy(p1_hbm.at[idxs.at[r]], pbufs[r],
                                  sems_g[r]).wait()
            pltpu.make_async_copy(q1_hbm.at[idxd.at[r]], qbufs[r],
                                  sems_g[r]).wait()
            pltpu.make_async_copy(r1_hbm.at[pl.ds(0, CHUNK)], rbufs[r],
                                  sems_g[r]).wait()

    def process(j, r):
        pbuf, qbuf, rbuf = pbufs[r], qbufs[r], rbufs[r]
        wait_gathers(r)

        def row(rw, cr):
            for g in range(HALF // 16):
                sl = pl.ds(g * 16, 16)
                pbuf[rw, sl] = jnp.maximum(
                    pbuf[rw, sl] + qbuf[rw, sl] + rbuf[rw, sl], 0.0)
            return cr

        lax.fori_loop(0, CHUNK, row, 0, unroll=2)
        base = s * EDGES_PER_TILE + j * CHUNK
        pltpu.async_copy(
            pbuf, enew_hbm.at[pl.ds(base, CHUNK), pl.ds(c * HALF, HALF)],
            sems_o[r])
        pltpu.async_copy(pbuf, acc.at[idxd.at[r]], sems_a[r], add=True)

    def drain_out(r):
        pltpu.make_async_copy(
            pbufs[r], enew_hbm.at[pl.ds(0, CHUNK), pl.ds(c * HALF, HALF)],
            sems_o[r]).wait()
        pltpu.make_async_copy(z_hbm.at[pl.ds(0, CHUNK)], pbufs[r],
                              sems_a[r]).wait()

    def slot(t, rt):
        @pl.when(t + 1 < N_CHUNKS)
        def _gathers_next():
            wait_idx((rt + 1) % NRING)
            issue_gathers(t + 1, (rt + 1) % NRING)

        @pl.when(t + 2 < N_CHUNKS)
        def _idx_ahead():
            @pl.when(t >= 1)
            def _drain():
                drain_out((rt + 2) % NRING)
            issue_idx(t + 2, (rt + 2) % NRING)

        @pl.when(t < N_CHUNKS)
        def _proc():
            process(t, rt)

    issue_idx(0, 0)
    issue_idx(1, 1)
    wait_idx(0)
    issue_gathers(0, 0)

    def triple(m, carry):
        slot(3 * m, 0)
        slot(3 * m + 1, 1)
        slot(3 * m + 2, 2)
        return carry

    lax.fori_loop(0, (N_CHUNKS + 2) // 3, triple, 0)
    drain_out(0)
    drain_out(1)
    drain_out(2)
    plsc.subcore_barrier()

    @pl.when(s < FLUSH_TILES)
    def _flush():
        pltpu.sync_copy(acc.at[pl.ds(s * FLUSH_ROWS, FLUSH_ROWS)],
                        agg_hbm.at[c, pl.ds(s * FLUSH_ROWS, FLUSH_ROWS)])


def _sc_edge_pass(src3, dst3, p0, p1, q0, q1, r0, r1, zeros):
    mesh = plsc.VectorSubcoreMesh(core_axis_name="c", subcore_axis_name="s")
    f = pl.kernel(
        _sc_body,
        mesh=mesh,
        out_type=[
            jax.ShapeDtypeStruct((N_EDGES, D_FEAT), jnp.float32),
            jax.ShapeDtypeStruct((2, N_NODES, HALF), jnp.float32),
        ],
        scratch_types=(
            [pltpu.VMEM((NRING, CHUNK), jnp.int32)] * 2
            + [pltpu.VMEM((CHUNK, HALF), jnp.float32)] * 9
            + [pltpu.VMEM_SHARED((N_NODES, HALF), jnp.float32)]
            + [pltpu.SemaphoreType.DMA] * 12
        ),
    )
    return f(src3, dst3, p0, p1, q0, q1, r0, r1, zeros)


def kernel(nodes, edges, edge_index, W_e, b_e, W_n, b_n):
    src3 = edge_index[0].reshape(NS, N_CHUNKS, CHUNK)
    dst3 = edge_index[1].reshape(NS, N_CHUNKS, CHUNK)
    we_sd = jnp.stack([W_e[:D_FEAT], W_e[D_FEAT:2 * D_FEAT]])
    we_e = W_e[2 * D_FEAT:]
    wn1 = W_n[:D_FEAT]
    wn2r = W_n[D_FEAT:].reshape(2, HALF, D_FEAT)

    pq = _compute_pq(nodes, we_sd)
    r3 = _compute_r(edges, we_e, b_e)
    zeros = jnp.zeros((N_NODES, HALF), jnp.float32)

    e_new, agg = _sc_edge_pass(src3, dst3, pq[0, 0], pq[0, 1],
                               pq[1, 0], pq[1, 1], r3[0], r3[1], zeros)
    n_new = _node_update(nodes, agg, wn1, wn2r, b_n)
    return (n_new, e_new)


# speedup vs baseline: 1.4286x; 1.4249x over previous
import jax
import jax.numpy as jnp
from jax import lax
from jax.experimental import pallas as pl
from jax.experimental.pallas import tpu as pltpu
from jax.experimental.pallas import tpu_sc as plsc

N_NODES = 10000
N_EDGES = 160000
D_FEAT = 256
HALF = 128

NC = 2
NS = 16
CHUNK = 80
N_CHUNKS = N_EDGES // CHUNK // NS
NRING = 4
FLUSH_TILES = 10
FLUSH_ROWS = N_NODES // FLUSH_TILES


def _pq_body(nodes_ref, w_ref, out_ref):
    out_ref[0, 0] = jnp.dot(nodes_ref[...], w_ref[0],
                            preferred_element_type=jnp.float32)


def _compute_pq(nodes, we_sd):
    blk = 2000
    grid = (N_NODES // blk, 2, 2)
    return pl.pallas_call(
        _pq_body,
        grid=grid,
        in_specs=[
            pl.BlockSpec((blk, D_FEAT), lambda i, w, c: (i, 0)),
            pl.BlockSpec((1, D_FEAT, HALF), lambda i, w, c: (w, 0, c)),
        ],
        out_specs=pl.BlockSpec((1, 1, blk, HALF), lambda i, w, c: (w, c, i, 0)),
        out_shape=jax.ShapeDtypeStruct((2, 2, N_NODES, HALF), jnp.float32),
    )(nodes, we_sd)


def _r_body(e_ref, w_ref, b_ref, out_ref):
    out_ref[0] = (jnp.dot(e_ref[...], w_ref[...],
                          preferred_element_type=jnp.float32)
                  + b_ref[...][None, :])


def _compute_r(edges, we_e, b_e):
    blk = 4000
    grid = (N_EDGES // blk, 2)
    return pl.pallas_call(
        _r_body,
        grid=grid,
        in_specs=[
            pl.BlockSpec((blk, 16), lambda j, c: (j, 0)),
            pl.BlockSpec((16, HALF), lambda j, c: (0, c)),
            pl.BlockSpec((HALF,), lambda j, c: (c,)),
        ],
        out_specs=pl.BlockSpec((1, blk, HALF), lambda j, c: (c, j, 0)),
        out_shape=jax.ShapeDtypeStruct((2, N_EDGES, HALF), jnp.float32),
    )(edges, we_e, b_e)


def _node_body(nodes_ref, agg_ref, wn1_ref, wn2_ref, b_ref, out_ref):
    acc = jnp.dot(nodes_ref[...], wn1_ref[...],
                  preferred_element_type=jnp.float32)
    acc += jnp.dot(agg_ref[0], wn2_ref[0], preferred_element_type=jnp.float32)
    acc += jnp.dot(agg_ref[1], wn2_ref[1], preferred_element_type=jnp.float32)
    out_ref[...] = jnp.maximum(acc + b_ref[...][None, :], 0.0)


def _node_update(nodes, agg, wn1, wn2r, b_n):
    blk = 2000
    grid = (N_NODES // blk, 2)
    return pl.pallas_call(
        _node_body,
        grid=grid,
        in_specs=[
            pl.BlockSpec((blk, D_FEAT), lambda i, h: (i, 0)),
            pl.BlockSpec((2, blk, HALF), lambda i, h: (0, i, 0)),
            pl.BlockSpec((D_FEAT, HALF), lambda i, h: (0, h)),
            pl.BlockSpec((2, HALF, HALF), lambda i, h: (0, 0, h)),
            pl.BlockSpec((HALF,), lambda i, h: (h,)),
        ],
        out_specs=pl.BlockSpec((blk, HALF), lambda i, h: (i, h)),
        out_shape=jax.ShapeDtypeStruct((N_NODES, D_FEAT), jnp.float32),
    )(nodes, agg, wn1, wn2r, b_n)


def _sc_body(src2_hbm, dst2_hbm, p0_hbm, p1_hbm, q0_hbm, q1_hbm,
             r0_hbm, r1_hbm, z_hbm,
             enew_hbm, agg_hbm,
             idxs, idxd, rv,
             pbuf0, pbuf1, pbuf2, pbuf3,
             acc,
             sem_i0, sem_i1, sem_i2, sem_i3,
             sem_g0, sem_g1, sem_g2, sem_g3,
             sem_o0, sem_o1, sem_o2, sem_o3,
             sem_a0, sem_a1, sem_a2, sem_a3):
    c = lax.axis_index("c")
    s = lax.axis_index("s")
    pbufs = (pbuf0, pbuf1, pbuf2, pbuf3)
    sems_i = (sem_i0, sem_i1, sem_i2, sem_i3)
    sems_g = (sem_g0, sem_g1, sem_g2, sem_g3)
    sems_o = (sem_o0, sem_o1, sem_o2, sem_o3)
    sems_a = (sem_a0, sem_a1, sem_a2, sem_a3)
    iota16 = lax.iota(jnp.int32, 16)

    @pl.when(s < FLUSH_TILES)
    def _zero():
        pltpu.sync_copy(z_hbm.at[pl.ds(s * FLUSH_ROWS, FLUSH_ROWS)],
                        acc.at[pl.ds(s * FLUSH_ROWS, FLUSH_ROWS)])
    plsc.subcore_barrier()

    def drain_g(r):
        pltpu.make_async_copy(z_hbm.at[pl.ds(0, CHUNK)], pbufs[r],
                              sems_g[r]).wait()

    def issue_idx(j, r):
        g = s + NS * j
        pltpu.async_copy(src2_hbm.at[g], idxs.at[r], sems_i[r])
        pltpu.async_copy(dst2_hbm.at[g], idxd.at[r], sems_i[r])

    def wait_idx(r):
        pltpu.make_async_copy(src2_hbm.at[0], idxs.at[r], sems_i[r]).wait()
        pltpu.make_async_copy(dst2_hbm.at[0], idxd.at[r], sems_i[r]).wait()

    def issue_p(r):
        @pl.when(c == 0)
        def _c0():
            pltpu.async_copy(p0_hbm.at[idxs.at[r]], pbufs[r], sems_g[r])

        @pl.when(c == 1)
        def _c1():
            pltpu.async_copy(p1_hbm.at[idxs.at[r]], pbufs[r], sems_g[r])

    def issue_q(r):
        @pl.when(c == 0)
        def _c0():
            pltpu.async_copy(q0_hbm.at[idxd.at[r]], pbufs[r], sems_g[r],
                             add=True)

        @pl.when(c == 1)
        def _c1():
            pltpu.async_copy(q1_hbm.at[idxd.at[r]], pbufs[r], sems_g[r],
                             add=True)

    def issue_r(j, r):
        base = (s + NS * j) * CHUNK
        for g in range(CHUNK // 16):
            rv[r, pl.ds(g * 16, 16)] = base + g * 16 + iota16

        @pl.when(c == 0)
        def _c0():
            pltpu.async_copy(r0_hbm.at[rv.at[r]], pbufs[r], sems_g[r],
                             add=True)

        @pl.when(c == 1)
        def _c1():
            pltpu.async_copy(r1_hbm.at[rv.at[r]], pbufs[r], sems_g[r],
                             add=True)

    def process(j, r):
        pbuf = pbufs[r]
        drain_g(r)

        def row(rw, cr):
            for g in range(HALF // 16):
                sl = pl.ds(g * 16, 16)
                pbuf[rw, sl] = jnp.maximum(pbuf[rw, sl], 0.0)
            return cr

        lax.fori_loop(0, CHUNK, row, 0, unroll=2)
        base = (s + NS * j) * CHUNK
        pltpu.async_copy(
            pbuf, enew_hbm.at[pl.ds(base, CHUNK), pl.ds(c * HALF, HALF)],
            sems_o[r])
        pltpu.async_copy(pbuf, acc.at[idxd.at[r]], sems_a[r], add=True)

    def drain_out(r):
        pltpu.make_async_copy(
            pbufs[r], enew_hbm.at[pl.ds(0, CHUNK), pl.ds(c * HALF, HALF)],
            sems_o[r]).wait()
        pltpu.make_async_copy(z_hbm.at[pl.ds(0, CHUNK)], pbufs[r],
                              sems_a[r]).wait()

    def slot(t, rt):
        @pl.when(t + 1 < N_CHUNKS)
        def _q_next():
            drain_g((rt + 1) % NRING)
            issue_q((rt + 1) % NRING)

        @pl.when(t + 2 < N_CHUNKS)
        def _p_ahead():
            wait_idx((rt + 2) % NRING)
            issue_p((rt + 2) % NRING)

        @pl.when(t + 3 < N_CHUNKS)
        def _idx_ahead():
            @pl.when(t >= 1)
            def _drain():
                drain_out((rt + 3) % NRING)
            issue_idx(t + 3, (rt + 3) % NRING)

        @pl.when(t < N_CHUNKS)
        def _proc():
            process(t, rt)

        @pl.when(t + 1 < N_CHUNKS)
        def _r_next():
            drain_g((rt + 1) % NRING)
            issue_r(t + 1, (rt + 1) % NRING)

    issue_idx(0, 0)
    issue_idx(1, 1)
    issue_idx(2, 2)
    wait_idx(0)
    issue_p(0)
    wait_idx(1)
    issue_p(1)
    drain_g(0)
    issue_q(0)
    drain_g(0)
    issue_r(0, 0)

    def quad(m, carry):
        slot(4 * m, 0)
        slot(4 * m + 1, 1)
        slot(4 * m + 2, 2)
        slot(4 * m + 3, 3)
        return carry

    lax.fori_loop(0, (N_CHUNKS + 3) // 4, quad, 0)
    drain_out(0)
    drain_out(1)
    drain_out(2)
    drain_out(3)
    plsc.subcore_barrier()

    @pl.when(s < FLUSH_TILES)
    def _flush():
        pltpu.sync_copy(acc.at[pl.ds(s * FLUSH_ROWS, FLUSH_ROWS)],
                        agg_hbm.at[c, pl.ds(s * FLUSH_ROWS, FLUSH_ROWS)])


def _sc_edge_pass(src2, dst2, p0, p1, q0, q1, r0, r1, zeros):
    mesh = plsc.VectorSubcoreMesh(core_axis_name="c", subcore_axis_name="s")
    f = pl.kernel(
        _sc_body,
        mesh=mesh,
        out_type=[
            jax.ShapeDtypeStruct((N_EDGES, D_FEAT), jnp.float32),
            jax.ShapeDtypeStruct((2, N_NODES, HALF), jnp.float32),
        ],
        scratch_types=(
            [pltpu.VMEM((NRING, CHUNK), jnp.int32)] * 3
            + [pltpu.VMEM((CHUNK, HALF), jnp.float32)] * NRING
            + [pltpu.VMEM_SHARED((N_NODES, HALF), jnp.float32)]
            + [pltpu.SemaphoreType.DMA] * 16
        ),
    )
    return f(src2, dst2, p0, p1, q0, q1, r0, r1, zeros)


def kernel(nodes, edges, edge_index, W_e, b_e, W_n, b_n):
    src2 = edge_index[0].reshape(N_EDGES // CHUNK, CHUNK)
    dst2 = edge_index[1].reshape(N_EDGES // CHUNK, CHUNK)
    we_sd = jnp.stack([W_e[:D_FEAT], W_e[D_FEAT:2 * D_FEAT]])
    we_e = W_e[2 * D_FEAT:]
    wn1 = W_n[:D_FEAT]
    wn2r = W_n[D_FEAT:].reshape(2, HALF, D_FEAT)

    pq = _compute_pq(nodes, we_sd)
    r3 = _compute_r(edges, we_e, b_e)
    zeros = jnp.zeros((N_NODES, HALF), jnp.float32)

    e_new, agg = _sc_edge_pass(src2, dst2, pq[0, 0], pq[0, 1],
                               pq[1, 0], pq[1, 1], r3[0], r3[1], zeros)
    n_new = _node_update(nodes, agg, wn1, wn2r, b_n)
    return (n_new, e_new)


# speedup vs baseline: 1.9304x; 1.3513x over previous
import jax
import jax.numpy as jnp
from jax import lax
from jax.experimental import pallas as pl
from jax.experimental.pallas import tpu as pltpu
from jax.experimental.pallas import tpu_sc as plsc

N_NODES = 10000
N_EDGES = 160000
D_FEAT = 256
HALF = 128

NC = 2
NS = 16
CHUNK = 80
N_CHUNKS = N_EDGES // CHUNK // NS
NRING = 4
FLUSH_TILES = 10
FLUSH_ROWS = N_NODES // FLUSH_TILES


def _pq_body(nodes_ref, w_ref, p0_ref, p1_ref, q0_ref, q1_ref):
    n = nodes_ref[...]
    ps = jnp.dot(n, w_ref[0], preferred_element_type=jnp.float32)
    qs = jnp.dot(n, w_ref[1], preferred_element_type=jnp.float32)
    p0_ref[...] = ps[:, :HALF]
    p1_ref[...] = ps[:, HALF:]
    q0_ref[...] = qs[:, :HALF]
    q1_ref[...] = qs[:, HALF:]


def _compute_pq(nodes, we_sd):
    blk = 2000
    grid = (N_NODES // blk,)
    half = jax.ShapeDtypeStruct((N_NODES, HALF), jnp.float32)
    return pl.pallas_call(
        _pq_body,
        grid=grid,
        in_specs=[
            pl.BlockSpec((blk, D_FEAT), lambda i: (i, 0)),
            pl.BlockSpec((2, D_FEAT, D_FEAT), lambda i: (0, 0, 0)),
        ],
        out_specs=[pl.BlockSpec((blk, HALF), lambda i: (i, 0))] * 4,
        out_shape=[half] * 4,
    )(nodes, we_sd)


def _r_body(e_ref, w_ref, b_ref, r0_ref, r1_ref):
    rr = (jnp.dot(e_ref[...], w_ref[...],
                  preferred_element_type=jnp.float32)
          + b_ref[...][None, :])
    r0_ref[...] = rr[:, :HALF]
    r1_ref[...] = rr[:, HALF:]


def _compute_r(edges, we_e, b_e):
    blk = 4000
    grid = (N_EDGES // blk,)
    half = jax.ShapeDtypeStruct((N_EDGES, HALF), jnp.float32)
    return pl.pallas_call(
        _r_body,
        grid=grid,
        in_specs=[
            pl.BlockSpec((blk, 16), lambda j: (j, 0)),
            pl.BlockSpec((16, D_FEAT), lambda j: (0, 0)),
            pl.BlockSpec((D_FEAT,), lambda j: (0,)),
        ],
        out_specs=[pl.BlockSpec((blk, HALF), lambda j: (j, 0))] * 2,
        out_shape=[half] * 2,
    )(edges, we_e, b_e)


def _node_body(nodes_ref, agg_ref, wn1_ref, wn2_ref, b_ref, out_ref):
    acc = jnp.dot(nodes_ref[...], wn1_ref[...],
                  preferred_element_type=jnp.float32)
    acc += jnp.dot(agg_ref[0], wn2_ref[0], preferred_element_type=jnp.float32)
    acc += jnp.dot(agg_ref[1], wn2_ref[1], preferred_element_type=jnp.float32)
    out_ref[...] = jnp.maximum(acc + b_ref[...][None, :], 0.0)


def _node_update(nodes, agg, wn1, wn2r, b_n):
    blk = 2000
    grid = (N_NODES // blk, 2)
    return pl.pallas_call(
        _node_body,
        grid=grid,
        in_specs=[
            pl.BlockSpec((blk, D_FEAT), lambda i, h: (i, 0)),
            pl.BlockSpec((2, blk, HALF), lambda i, h: (0, i, 0)),
            pl.BlockSpec((D_FEAT, HALF), lambda i, h: (0, h)),
            pl.BlockSpec((2, HALF, HALF), lambda i, h: (0, 0, h)),
            pl.BlockSpec((HALF,), lambda i, h: (h,)),
        ],
        out_specs=pl.BlockSpec((blk, HALF), lambda i, h: (i, h)),
        out_shape=jax.ShapeDtypeStruct((N_NODES, D_FEAT), jnp.float32),
    )(nodes, agg, wn1, wn2r, b_n)


def _sc_body(src2_hbm, dst2_hbm, p0_hbm, p1_hbm, q0_hbm, q1_hbm,
             r0_hbm, r1_hbm, z_hbm,
             enew_hbm, agg_hbm,
             idxs, idxd, rv,
             pbuf0, pbuf1, pbuf2, pbuf3,
             acc,
             sem_i0, sem_i1, sem_i2, sem_i3,
             sem_g0, sem_g1, sem_g2, sem_g3,
             sem_o0, sem_o1, sem_o2, sem_o3,
             sem_a0, sem_a1, sem_a2, sem_a3):
    c = lax.axis_index("c")
    s = lax.axis_index("s")
    pbufs = (pbuf0, pbuf1, pbuf2, pbuf3)
    sems_i = (sem_i0, sem_i1, sem_i2, sem_i3)
    sems_g = (sem_g0, sem_g1, sem_g2, sem_g3)
    sems_o = (sem_o0, sem_o1, sem_o2, sem_o3)
    sems_a = (sem_a0, sem_a1, sem_a2, sem_a3)
    iota16 = lax.iota(jnp.int32, 16)

    @pl.when(s < FLUSH_TILES)
    def _zero():
        pltpu.sync_copy(z_hbm.at[pl.ds(s * FLUSH_ROWS, FLUSH_ROWS)],
                        acc.at[pl.ds(s * FLUSH_ROWS, FLUSH_ROWS)])
    plsc.subcore_barrier()

    def drain_g(r):
        pltpu.make_async_copy(z_hbm.at[pl.ds(0, CHUNK)], pbufs[r],
                              sems_g[r]).wait()

    def issue_idx(j, r):
        g = s + NS * j
        pltpu.async_copy(src2_hbm.at[g], idxs.at[r], sems_i[r])
        pltpu.async_copy(dst2_hbm.at[g], idxd.at[r], sems_i[r])

    def wait_idx(r):
        pltpu.make_async_copy(src2_hbm.at[0], idxs.at[r], sems_i[r]).wait()
        pltpu.make_async_copy(dst2_hbm.at[0], idxd.at[r], sems_i[r]).wait()

    def issue_p(r):
        @pl.when(c == 0)
        def _c0():
            pltpu.async_copy(p0_hbm.at[idxs.at[r]], pbufs[r], sems_g[r])

        @pl.when(c == 1)
        def _c1():
            pltpu.async_copy(p1_hbm.at[idxs.at[r]], pbufs[r], sems_g[r])

    def issue_q(r):
        @pl.when(c == 0)
        def _c0():
            pltpu.async_copy(q0_hbm.at[idxd.at[r]], pbufs[r], sems_g[r],
                             add=True)

        @pl.when(c == 1)
        def _c1():
            pltpu.async_copy(q1_hbm.at[idxd.at[r]], pbufs[r], sems_g[r],
                             add=True)

    def issue_r(j, r):
        base = (s + NS * j) * CHUNK
        for g in range(CHUNK // 16):
            rv[r, pl.ds(g * 16, 16)] = base + g * 16 + iota16

        @pl.when(c == 0)
        def _c0():
            pltpu.async_copy(r0_hbm.at[rv.at[r]], pbufs[r], sems_g[r],
                             add=True)

        @pl.when(c == 1)
        def _c1():
            pltpu.async_copy(r1_hbm.at[rv.at[r]], pbufs[r], sems_g[r],
                             add=True)

    def process(j, r):
        pbuf = pbufs[r]
        drain_g(r)

        def row(rw, cr):
            for g in range(HALF // 16):
                sl = pl.ds(g * 16, 16)
                pbuf[rw, sl] = jnp.maximum(pbuf[rw, sl], 0.0)
            return cr

        lax.fori_loop(0, CHUNK, row, 0, unroll=2)
        base = (s + NS * j) * CHUNK
        pltpu.async_copy(
            pbuf, enew_hbm.at[pl.ds(base, CHUNK), pl.ds(c * HALF, HALF)],
            sems_o[r])
        pltpu.async_copy(pbuf, acc.at[idxd.at[r]], sems_a[r], add=True)

    def drain_out(r):
        pltpu.make_async_copy(
            pbufs[r], enew_hbm.at[pl.ds(0, CHUNK), pl.ds(c * HALF, HALF)],
            sems_o[r]).wait()
        pltpu.make_async_copy(z_hbm.at[pl.ds(0, CHUNK)], pbufs[r],
                              sems_a[r]).wait()

    def slot(t, rt):
        @pl.when(t + 1 < N_CHUNKS)
        def _q_next():
            drain_g((rt + 1) % NRING)
            issue_q((rt + 1) % NRING)

        @pl.when(t + 2 < N_CHUNKS)
        def _p_ahead():
            wait_idx((rt + 2) % NRING)
            issue_p((rt + 2) % NRING)

        @pl.when(t + 3 < N_CHUNKS)
        def _idx_ahead():
            @pl.when(t >= 1)
            def _drain():
                drain_out((rt + 3) % NRING)
            issue_idx(t + 3, (rt + 3) % NRING)

        @pl.when(t < N_CHUNKS)
        def _proc():
            process(t, rt)

        @pl.when(t + 1 < N_CHUNKS)
        def _r_next():
            drain_g((rt + 1) % NRING)
            issue_r(t + 1, (rt + 1) % NRING)

    issue_idx(0, 0)
    issue_idx(1, 1)
    issue_idx(2, 2)
    wait_idx(0)
    issue_p(0)
    wait_idx(1)
    issue_p(1)
    drain_g(0)
    issue_q(0)
    drain_g(0)
    issue_r(0, 0)

    def quad(m, carry):
        slot(4 * m, 0)
        slot(4 * m + 1, 1)
        slot(4 * m + 2, 2)
        slot(4 * m + 3, 3)
        return carry

    lax.fori_loop(0, (N_CHUNKS + 3) // 4, quad, 0)
    drain_out(0)
    drain_out(1)
    drain_out(2)
    drain_out(3)
    plsc.subcore_barrier()

    @pl.when(s < FLUSH_TILES)
    def _flush():
        pltpu.sync_copy(acc.at[pl.ds(s * FLUSH_ROWS, FLUSH_ROWS)],
                        agg_hbm.at[c, pl.ds(s * FLUSH_ROWS, FLUSH_ROWS)])


def _sc_edge_pass(src2, dst2, p0, p1, q0, q1, r0, r1, zeros):
    mesh = plsc.VectorSubcoreMesh(core_axis_name="c", subcore_axis_name="s")
    f = pl.kernel(
        _sc_body,
        mesh=mesh,
        out_type=[
            jax.ShapeDtypeStruct((N_EDGES, D_FEAT), jnp.float32),
            jax.ShapeDtypeStruct((2, N_NODES, HALF), jnp.float32),
        ],
        scratch_types=(
            [pltpu.VMEM((NRING, CHUNK), jnp.int32)] * 3
            + [pltpu.VMEM((CHUNK, HALF), jnp.float32)] * NRING
            + [pltpu.VMEM_SHARED((N_NODES, HALF), jnp.float32)]
            + [pltpu.SemaphoreType.DMA] * 16
        ),
    )
    return f(src2, dst2, p0, p1, q0, q1, r0, r1, zeros)


def kernel(nodes, edges, edge_index, W_e, b_e, W_n, b_n):
    src2 = edge_index[0].reshape(N_EDGES // CHUNK, CHUNK)
    dst2 = edge_index[1].reshape(N_EDGES // CHUNK, CHUNK)
    we_sd = jnp.stack([W_e[:D_FEAT], W_e[D_FEAT:2 * D_FEAT]])
    we_e = W_e[2 * D_FEAT:]
    wn1 = W_n[:D_FEAT]
    wn2r = W_n[D_FEAT:].reshape(2, HALF, D_FEAT)

    p0, p1, q0, q1 = _compute_pq(nodes, we_sd)
    r0, r1 = _compute_r(edges, we_e, b_e)
    zeros = jnp.zeros((N_NODES, HALF), jnp.float32)

    e_new, agg = _sc_edge_pass(src2, dst2, p0, p1, q0, q1, r0, r1, zeros)
    n_new = _node_update(nodes, agg, wn1, wn2r, b_n)
    return (n_new, e_new)


# speedup vs baseline: 1.9474x; 1.0088x over previous
import jax
import jax.numpy as jnp
from jax import lax
from jax.experimental import pallas as pl
from jax.experimental.pallas import tpu as pltpu
from jax.experimental.pallas import tpu_sc as plsc

N_NODES = 10000
N_EDGES = 160000
D_FEAT = 256
HALF = 128

NC = 2
NS = 16
CHUNK = 80
N_CHUNKS = N_EDGES // CHUNK // NS
NRING = 4
FLUSH_TILES = 10
FLUSH_ROWS = N_NODES // FLUSH_TILES


_NBLK = 2000
_EBLK = 4000
_NPQ = N_NODES // _NBLK


def _pqr_body(nodes_ref, w_ref, e_ref, we_ref, b_ref,
              p0_ref, p1_ref, q0_ref, q1_ref, r0_ref, r1_ref):
    j = pl.program_id(0)

    @pl.when(j < _NPQ)
    def _pq():
        n = nodes_ref[...]
        ps = jnp.dot(n, w_ref[0], preferred_element_type=jnp.float32)
        qs = jnp.dot(n, w_ref[1], preferred_element_type=jnp.float32)
        p0_ref[...] = ps[:, :HALF]
        p1_ref[...] = ps[:, HALF:]
        q0_ref[...] = qs[:, :HALF]
        q1_ref[...] = qs[:, HALF:]

    rr = (jnp.dot(e_ref[...], we_ref[...],
                  preferred_element_type=jnp.float32)
          + b_ref[...][None, :])
    r0_ref[...] = rr[:, :HALF]
    r1_ref[...] = rr[:, HALF:]


def _compute_pqr(nodes, we_sd, edges, we_e, b_e):
    grid = (N_EDGES // _EBLK,)
    nhalf = jax.ShapeDtypeStruct((N_NODES, HALF), jnp.float32)
    ehalf = jax.ShapeDtypeStruct((N_EDGES, HALF), jnp.float32)

    def _pq_idx(j):
        return (jnp.minimum(j, _NPQ - 1), 0)

    return pl.pallas_call(
        _pqr_body,
        grid=grid,
        in_specs=[
            pl.BlockSpec((_NBLK, D_FEAT), _pq_idx),
            pl.BlockSpec((2, D_FEAT, D_FEAT), lambda j: (0, 0, 0)),
            pl.BlockSpec((_EBLK, 16), lambda j: (j, 0)),
            pl.BlockSpec((16, D_FEAT), lambda j: (0, 0)),
            pl.BlockSpec((D_FEAT,), lambda j: (0,)),
        ],
        out_specs=([pl.BlockSpec((_NBLK, HALF), _pq_idx)] * 4
                   + [pl.BlockSpec((_EBLK, HALF), lambda j: (j, 0))] * 2),
        out_shape=[nhalf] * 4 + [ehalf] * 2,
    )(nodes, we_sd, edges, we_e, b_e)


def _node_body(nodes_ref, agg_ref, wn1_ref, wn2_ref, b_ref, out_ref):
    acc = jnp.dot(nodes_ref[...], wn1_ref[...],
                  preferred_element_type=jnp.float32)
    acc += jnp.dot(agg_ref[0], wn2_ref[0], preferred_element_type=jnp.float32)
    acc += jnp.dot(agg_ref[1], wn2_ref[1], preferred_element_type=jnp.float32)
    out_ref[...] = jnp.maximum(acc + b_ref[...][None, :], 0.0)


def _node_update(nodes, agg, wn1, wn2r, b_n):
    blk = 2000
    grid = (N_NODES // blk, 2)
    return pl.pallas_call(
        _node_body,
        grid=grid,
        in_specs=[
            pl.BlockSpec((blk, D_FEAT), lambda i, h: (i, 0)),
            pl.BlockSpec((2, blk, HALF), lambda i, h: (0, i, 0)),
            pl.BlockSpec((D_FEAT, HALF), lambda i, h: (0, h)),
            pl.BlockSpec((2, HALF, HALF), lambda i, h: (0, 0, h)),
            pl.BlockSpec((HALF,), lambda i, h: (h,)),
        ],
        out_specs=pl.BlockSpec((blk, HALF), lambda i, h: (i, h)),
        out_shape=jax.ShapeDtypeStruct((N_NODES, D_FEAT), jnp.float32),
    )(nodes, agg, wn1, wn2r, b_n)


def _sc_body(src2_hbm, dst2_hbm, p0_hbm, p1_hbm, q0_hbm, q1_hbm,
             r0_hbm, r1_hbm, z_hbm,
             enew_hbm, agg_hbm,
             idxs, idxd, rv,
             pbuf0, pbuf1, pbuf2, pbuf3,
             acc,
             sem_i0, sem_i1, sem_i2, sem_i3,
             sem_g0, sem_g1, sem_g2, sem_g3,
             sem_o0, sem_o1, sem_o2, sem_o3,
             sem_a0, sem_a1, sem_a2, sem_a3):
    c = lax.axis_index("c")
    s = lax.axis_index("s")
    pbufs = (pbuf0, pbuf1, pbuf2, pbuf3)
    sems_i = (sem_i0, sem_i1, sem_i2, sem_i3)
    sems_g = (sem_g0, sem_g1, sem_g2, sem_g3)
    sems_o = (sem_o0, sem_o1, sem_o2, sem_o3)
    sems_a = (sem_a0, sem_a1, sem_a2, sem_a3)
    iota16 = lax.iota(jnp.int32, 16)

    @pl.when(s < FLUSH_TILES)
    def _zero():
        pltpu.sync_copy(z_hbm.at[pl.ds(s * FLUSH_ROWS, FLUSH_ROWS)],
                        acc.at[pl.ds(s * FLUSH_ROWS, FLUSH_ROWS)])
    plsc.subcore_barrier()

    def drain_g(r):
        pltpu.make_async_copy(z_hbm.at[pl.ds(0, CHUNK)], pbufs[r],
                              sems_g[r]).wait()

    def issue_idx(j, r):
        g = s + NS * j
        pltpu.async_copy(src2_hbm.at[g], idxs.at[r], sems_i[r])
        pltpu.async_copy(dst2_hbm.at[g], idxd.at[r], sems_i[r])

    def wait_idx(r):
        pltpu.make_async_copy(src2_hbm.at[0], idxs.at[r], sems_i[r]).wait()
        pltpu.make_async_copy(dst2_hbm.at[0], idxd.at[r], sems_i[r]).wait()

    def issue_p(r):
        @pl.when(c == 0)
        def _c0():
            pltpu.async_copy(p0_hbm.at[idxs.at[r]], pbufs[r], sems_g[r])

        @pl.when(c == 1)
        def _c1():
            pltpu.async_copy(p1_hbm.at[idxs.at[r]], pbufs[r], sems_g[r])

    def issue_q(r):
        @pl.when(c == 0)
        def _c0():
            pltpu.async_copy(q0_hbm.at[idxd.at[r]], pbufs[r], sems_g[r],
                             add=True)

        @pl.when(c == 1)
        def _c1():
            pltpu.async_copy(q1_hbm.at[idxd.at[r]], pbufs[r], sems_g[r],
                             add=True)

    def issue_r(j, r):
        base = (s + NS * j) * CHUNK
        for g in range(CHUNK // 16):
            rv[r, pl.ds(g * 16, 16)] = base + g * 16 + iota16

        @pl.when(c == 0)
        def _c0():
            pltpu.async_copy(r0_hbm.at[rv.at[r]], pbufs[r], sems_g[r],
                             add=True)

        @pl.when(c == 1)
        def _c1():
            pltpu.async_copy(r1_hbm.at[rv.at[r]], pbufs[r], sems_g[r],
                             add=True)

    def process(j, r):
        pbuf = pbufs[r]
        drain_g(r)

        def row(rw, cr):
            for g in range(HALF // 16):
                sl = pl.ds(g * 16, 16)
                pbuf[rw, sl] = jnp.maximum(pbuf[rw, sl], 0.0)
            return cr

        lax.fori_loop(0, CHUNK, row, 0, unroll=2)
        base = (s + NS * j) * CHUNK
        pltpu.async_copy(
            pbuf, enew_hbm.at[pl.ds(base, CHUNK), pl.ds(c * HALF, HALF)],
            sems_o[r])
        pltpu.async_copy(pbuf, acc.at[idxd.at[r]], sems_a[r], add=True)

    def drain_out(r):
        pltpu.make_async_copy(
            pbufs[r], enew_hbm.at[pl.ds(0, CHUNK), pl.ds(c * HALF, HALF)],
            sems_o[r]).wait()
        pltpu.make_async_copy(z_hbm.at[pl.ds(0, CHUNK)], pbufs[r],
                              sems_a[r]).wait()

    def slot(t, rt):
        @pl.when(t + 1 < N_CHUNKS)
        def _q_next():
            drain_g((rt + 1) % NRING)
            issue_q((rt + 1) % NRING)

        @pl.when(t + 2 < N_CHUNKS)
        def _p_ahead():
            wait_idx((rt + 2) % NRING)
            issue_p((rt + 2) % NRING)

        @pl.when(t + 3 < N_CHUNKS)
        def _idx_ahead():
            @pl.when(t >= 1)
            def _drain():
                drain_out((rt + 3) % NRING)
            issue_idx(t + 3, (rt + 3) % NRING)

        @pl.when(t < N_CHUNKS)
        def _proc():
            process(t, rt)

        @pl.when(t + 1 < N_CHUNKS)
        def _r_next():
            drain_g((rt + 1) % NRING)
            issue_r(t + 1, (rt + 1) % NRING)

    issue_idx(0, 0)
    issue_idx(1, 1)
    issue_idx(2, 2)
    wait_idx(0)
    issue_p(0)
    wait_idx(1)
    issue_p(1)
    drain_g(0)
    issue_q(0)
    drain_g(0)
    issue_r(0, 0)

    def quad(m, carry):
        slot(4 * m, 0)
        slot(4 * m + 1, 1)
        slot(4 * m + 2, 2)
        slot(4 * m + 3, 3)
        return carry

    lax.fori_loop(0, (N_CHUNKS + 3) // 4, quad, 0)
    drain_out(0)
    drain_out(1)
    drain_out(2)
    drain_out(3)
    plsc.subcore_barrier()

    @pl.when(s < FLUSH_TILES)
    def _flush():
        pltpu.sync_copy(acc.at[pl.ds(s * FLUSH_ROWS, FLUSH_ROWS)],
                        agg_hbm.at[c, pl.ds(s * FLUSH_ROWS, FLUSH_ROWS)])


def _sc_edge_pass(src2, dst2, p0, p1, q0, q1, r0, r1, zeros):
    mesh = plsc.VectorSubcoreMesh(core_axis_name="c", subcore_axis_name="s")
    f = pl.kernel(
        _sc_body,
        mesh=mesh,
        out_type=[
            jax.ShapeDtypeStruct((N_EDGES, D_FEAT), jnp.float32),
            jax.ShapeDtypeStruct((2, N_NODES, HALF), jnp.float32),
        ],
        scratch_types=(
            [pltpu.VMEM((NRING, CHUNK), jnp.int32)] * 3
            + [pltpu.VMEM((CHUNK, HALF), jnp.float32)] * NRING
            + [pltpu.VMEM_SHARED((N_NODES, HALF), jnp.float32)]
            + [pltpu.SemaphoreType.DMA] * 16
        ),
    )
    return f(src2, dst2, p0, p1, q0, q1, r0, r1, zeros)


def kernel(nodes, edges, edge_index, W_e, b_e, W_n, b_n):
    src2 = edge_index[0].reshape(N_EDGES // CHUNK, CHUNK)
    dst2 = edge_index[1].reshape(N_EDGES // CHUNK, CHUNK)
    we_sd = jnp.stack([W_e[:D_FEAT], W_e[D_FEAT:2 * D_FEAT]])
    we_e = W_e[2 * D_FEAT:]
    wn1 = W_n[:D_FEAT]
    wn2r = W_n[D_FEAT:].reshape(2, HALF, D_FEAT)

    p0, p1, q0, q1, r0, r1 = _compute_pqr(nodes, we_sd, edges, we_e, b_e)
    zeros = jnp.zeros((N_NODES, HALF), jnp.float32)

    e_new, agg = _sc_edge_pass(src2, dst2, p0, p1, q0, q1, r0, r1, zeros)
    n_new = _node_update(nodes, agg, wn1, wn2r, b_n)
    return (n_new, e_new)


# speedup vs baseline: 1.9501x; 1.0014x over previous
import jax
import jax.numpy as jnp
from jax import lax
from jax.experimental import pallas as pl
from jax.experimental.pallas import tpu as pltpu
from jax.experimental.pallas import tpu_sc as plsc

N_NODES = 10000
N_EDGES = 160000
D_FEAT = 256
HALF = 128

NC = 2
NS = 16
CHUNK = 80
N_CHUNKS = N_EDGES // CHUNK // NS
NRING = 4
FLUSH_TILES = 10
FLUSH_ROWS = N_NODES // FLUSH_TILES


_NBLK = 2000
_EBLK = 4000
_NPQ = N_NODES // _NBLK


def _pqr_body(nodes_ref, w_ref, e_ref, we_ref, b_ref,
              p0_ref, p1_ref, q0_ref, q1_ref, r0_ref, r1_ref):
    j = pl.program_id(0)

    @pl.when(j < _NPQ)
    def _pq():
        n = nodes_ref[...]
        ps = jnp.dot(n, w_ref[0], preferred_element_type=jnp.float32)
        qs = jnp.dot(n, w_ref[1], preferred_element_type=jnp.float32)
        p0_ref[...] = ps[:, :HALF]
        p1_ref[...] = ps[:, HALF:]
        q0_ref[...] = qs[:, :HALF]
        q1_ref[...] = qs[:, HALF:]

    rr = (jnp.dot(e_ref[...], we_ref[...],
                  preferred_element_type=jnp.float32)
          + b_ref[...][None, :])
    r0_ref[...] = rr[:, :HALF]
    r1_ref[...] = rr[:, HALF:]


def _compute_pqr(nodes, we_sd, edges, we_e, b_e):
    grid = (N_EDGES // _EBLK,)
    nhalf = jax.ShapeDtypeStruct((N_NODES, HALF), jnp.float32)
    ehalf = jax.ShapeDtypeStruct((N_EDGES, HALF), jnp.float32)

    def _pq_idx(j):
        return (jnp.minimum(j, _NPQ - 1), 0)

    return pl.pallas_call(
        _pqr_body,
        grid=grid,
        in_specs=[
            pl.BlockSpec((_NBLK, D_FEAT), _pq_idx),
            pl.BlockSpec((2, D_FEAT, D_FEAT), lambda j: (0, 0, 0)),
            pl.BlockSpec((_EBLK, 16), lambda j: (j, 0)),
            pl.BlockSpec((16, D_FEAT), lambda j: (0, 0)),
            pl.BlockSpec((D_FEAT,), lambda j: (0,)),
        ],
        out_specs=([pl.BlockSpec((_NBLK, HALF), _pq_idx)] * 4
                   + [pl.BlockSpec((_EBLK, HALF), lambda j: (j, 0))] * 2),
        out_shape=[nhalf] * 4 + [ehalf] * 2,
    )(nodes, we_sd, edges, we_e, b_e)


def _node_body(nodes_ref, agg_ref, wn1_ref, wn2_ref, b_ref, out_ref):
    acc = jnp.dot(nodes_ref[...], wn1_ref[...],
                  preferred_element_type=jnp.float32)
    acc += jnp.dot(agg_ref[0], wn2_ref[0], preferred_element_type=jnp.float32)
    acc += jnp.dot(agg_ref[1], wn2_ref[1], preferred_element_type=jnp.float32)
    out_ref[...] = jnp.maximum(acc + b_ref[...][None, :], 0.0)


def _node_update(nodes, agg, wn1, wn2r, b_n):
    blk = 2000
    grid = (N_NODES // blk, 2)
    return pl.pallas_call(
        _node_body,
        grid=grid,
        in_specs=[
            pl.BlockSpec((blk, D_FEAT), lambda i, h: (i, 0)),
            pl.BlockSpec((2, blk, HALF), lambda i, h: (0, i, 0)),
            pl.BlockSpec((D_FEAT, HALF), lambda i, h: (0, h)),
            pl.BlockSpec((2, HALF, HALF), lambda i, h: (0, 0, h)),
            pl.BlockSpec((HALF,), lambda i, h: (h,)),
        ],
        out_specs=pl.BlockSpec((blk, HALF), lambda i, h: (i, h)),
        out_shape=jax.ShapeDtypeStruct((N_NODES, D_FEAT), jnp.float32),
    )(nodes, agg, wn1, wn2r, b_n)


def _sc_body(src2_hbm, dst2_hbm, p0_hbm, p1_hbm, q0_hbm, q1_hbm,
             r0_hbm, r1_hbm, z_hbm,
             enew_hbm, agg_hbm,
             idxs, idxd, rv,
             pbuf0, pbuf1, pbuf2, pbuf3,
             acc,
             sem_i0, sem_i1, sem_i2, sem_i3,
             sem_g0, sem_g1, sem_g2, sem_g3,
             sem_o0, sem_o1, sem_o2, sem_o3,
             sem_a0, sem_a1, sem_a2, sem_a3):
    c = lax.axis_index("c")
    s = lax.axis_index("s")
    pbufs = (pbuf0, pbuf1, pbuf2, pbuf3)
    sems_i = (sem_i0, sem_i1, sem_i2, sem_i3)
    sems_g = (sem_g0, sem_g1, sem_g2, sem_g3)
    sems_o = (sem_o0, sem_o1, sem_o2, sem_o3)
    sems_a = (sem_a0, sem_a1, sem_a2, sem_a3)
    iota16 = lax.iota(jnp.int32, 16)

    @pl.when(s < FLUSH_TILES)
    def _zero():
        pltpu.sync_copy(z_hbm.at[pl.ds(s * FLUSH_ROWS, FLUSH_ROWS)],
                        acc.at[pl.ds(s * FLUSH_ROWS, FLUSH_ROWS)])
    plsc.subcore_barrier()

    def drain_g(r):
        pltpu.make_async_copy(z_hbm.at[pl.ds(0, CHUNK)], pbufs[r],
                              sems_g[r]).wait()

    def issue_idx(j, r):
        g = s + NS * j
        pltpu.async_copy(src2_hbm.at[g], idxs.at[r], sems_i[r])
        pltpu.async_copy(dst2_hbm.at[g], idxd.at[r], sems_i[r])

    def wait_idx(r):
        pltpu.make_async_copy(src2_hbm.at[0], idxs.at[r], sems_i[r]).wait()
        pltpu.make_async_copy(dst2_hbm.at[0], idxd.at[r], sems_i[r]).wait()

    def issue_p(r):
        @pl.when(c == 0)
        def _c0():
            pltpu.async_copy(p0_hbm.at[idxs.at[r]], pbufs[r], sems_g[r])

        @pl.when(c == 1)
        def _c1():
            pltpu.async_copy(p1_hbm.at[idxs.at[r]], pbufs[r], sems_g[r])

    def issue_q(r):
        @pl.when(c == 0)
        def _c0():
            pltpu.async_copy(q0_hbm.at[idxd.at[r]], pbufs[r], sems_g[r],
                             add=True)

        @pl.when(c == 1)
        def _c1():
            pltpu.async_copy(q1_hbm.at[idxd.at[r]], pbufs[r], sems_g[r],
                             add=True)

    def issue_r(j, r):
        base = (s + NS * j) * CHUNK
        for g in range(CHUNK // 16):
            rv[r, pl.ds(g * 16, 16)] = base + g * 16 + iota16

        @pl.when(c == 0)
        def _c0():
            pltpu.async_copy(r0_hbm.at[rv.at[r]], pbufs[r], sems_g[r],
                             add=True)

        @pl.when(c == 1)
        def _c1():
            pltpu.async_copy(r1_hbm.at[rv.at[r]], pbufs[r], sems_g[r],
                             add=True)

    def process(j, r):
        pbuf = pbufs[r]
        drain_g(r)

        def row(rw, cr):
            for g in range(HALF // 16):
                sl = pl.ds(g * 16, 16)
                pbuf[rw, sl] = jnp.maximum(pbuf[rw, sl], 0.0)
            return cr

        lax.fori_loop(0, CHUNK, row, 0, unroll=4)
        base = (s + NS * j) * CHUNK
        pltpu.async_copy(
            pbuf, enew_hbm.at[pl.ds(base, CHUNK), pl.ds(c * HALF, HALF)],
            sems_o[r])
        pltpu.async_copy(pbuf, acc.at[idxd.at[r]], sems_a[r], add=True)

    def drain_out(r):
        pltpu.make_async_copy(
            pbufs[r], enew_hbm.at[pl.ds(0, CHUNK), pl.ds(c * HALF, HALF)],
            sems_o[r]).wait()
        pltpu.make_async_copy(z_hbm.at[pl.ds(0, CHUNK)], pbufs[r],
                              sems_a[r]).wait()

    def slot(t, rt):
        @pl.when(t + 1 < N_CHUNKS)
        def _q_next():
            drain_g((rt + 1) % NRING)
            issue_q((rt + 1) % NRING)

        @pl.when(t + 2 < N_CHUNKS)
        def _p_ahead():
            wait_idx((rt + 2) % NRING)
            issue_p((rt + 2) % NRING)

        @pl.when(t + 3 < N_CHUNKS)
        def _idx_ahead():
            @pl.when(t >= 1)
            def _drain():
                drain_out((rt + 3) % NRING)
            issue_idx(t + 3, (rt + 3) % NRING)

        @pl.when(t < N_CHUNKS)
        def _proc():
            process(t, rt)

        @pl.when(t + 1 < N_CHUNKS)
        def _r_next():
            drain_g((rt + 1) % NRING)
            issue_r(t + 1, (rt + 1) % NRING)

    issue_idx(0, 0)
    issue_idx(1, 1)
    issue_idx(2, 2)
    wait_idx(0)
    issue_p(0)
    wait_idx(1)
    issue_p(1)
    drain_g(0)
    issue_q(0)
    drain_g(0)
    issue_r(0, 0)

    def quad(m, carry):
        slot(4 * m, 0)
        slot(4 * m + 1, 1)
        slot(4 * m + 2, 2)
        slot(4 * m + 3, 3)
        return carry

    lax.fori_loop(0, (N_CHUNKS + 3) // 4, quad, 0)
    drain_out(0)
    drain_out(1)
    drain_out(2)
    drain_out(3)
    plsc.subcore_barrier()

    @pl.when(s < FLUSH_TILES)
    def _flush():
        pltpu.sync_copy(acc.at[pl.ds(s * FLUSH_ROWS, FLUSH_ROWS)],
                        agg_hbm.at[c, pl.ds(s * FLUSH_ROWS, FLUSH_ROWS)])


def _sc_edge_pass(src2, dst2, p0, p1, q0, q1, r0, r1, zeros):
    mesh = plsc.VectorSubcoreMesh(core_axis_name="c", subcore_axis_name="s")
    f = pl.kernel(
        _sc_body,
        mesh=mesh,
        out_type=[
            jax.ShapeDtypeStruct((N_EDGES, D_FEAT), jnp.float32),
            jax.ShapeDtypeStruct((2, N_NODES, HALF), jnp.float32),
        ],
        scratch_types=(
            [pltpu.VMEM((NRING, CHUNK), jnp.int32)] * 3
            + [pltpu.VMEM((CHUNK, HALF), jnp.float32)] * NRING
            + [pltpu.VMEM_SHARED((N_NODES, HALF), jnp.float32)]
            + [pltpu.SemaphoreType.DMA] * 16
        ),
    )
    return f(src2, dst2, p0, p1, q0, q1, r0, r1, zeros)


def kernel(nodes, edges, edge_index, W_e, b_e, W_n, b_n):
    src2 = edge_index[0].reshape(N_EDGES // CHUNK, CHUNK)
    dst2 = edge_index[1].reshape(N_EDGES // CHUNK, CHUNK)
    we_sd = jnp.stack([W_e[:D_FEAT], W_e[D_FEAT:2 * D_FEAT]])
    we_e = W_e[2 * D_FEAT:]
    wn1 = W_n[:D_FEAT]
    wn2r = W_n[D_FEAT:].reshape(2, HALF, D_FEAT)

    p0, p1, q0, q1, r0, r1 = _compute_pqr(nodes, we_sd, edges, we_e, b_e)
    zeros = jnp.zeros((N_NODES, HALF), jnp.float32)

    e_new, agg = _sc_edge_pass(src2, dst2, p0, p1, q0, q1, r0, r1, zeros)
    n_new = _node_update(nodes, agg, wn1, wn2r, b_n)
    return (n_new, e_new)
